# baseline (speedup 1.0000x reference)
import functools

import jax
import jax.numpy as jnp
from jax import lax
from jax.experimental import pallas as pl
from jax.experimental.pallas import tpu as pltpu
from jax.experimental.pallas import tpu_sc as plsc

N_NODES = 100000
D = 128
B = 10000
DEG = 16
ESZ = 3

NC = 2
NS = 16
NW = NC * NS

C_NODES = 2
ROWS_PER_CHUNK = C_NODES * DEG * ESZ
B_PAD = 10240
NODES_PER_W = B_PAD // NW
CHUNKS_PER_W = NODES_PER_W // C_NODES
NBUF = 4
DP = D // 32


def _body(table_hbm, idx_hbm, w_hbm, b_hbm, out_hbm,
          idx_stage, rows0, rows1, rows2, rows3, wv, bv, out_stage,
          sem0, sem1, sem2, sem3):
    rows_bufs = (rows0, rows1, rows2, rows3)
    sems = (sem0, sem1, sem2, sem3)
    wid = lax.axis_index("s") * NC + lax.axis_index("c")

    pltpu.sync_copy(w_hbm, wv)
    pltpu.sync_copy(b_hbm, bv)
    pltpu.sync_copy(idx_hbm.at[pl.ds(wid * CHUNKS_PER_W, CHUNKS_PER_W)],
                    idx_stage)
    wvec = wv[...] * jnp.float32(1.0 / (9.0 * D))
    bvec = bv[...]

    def start_gather(g, slot):
        pltpu.make_async_copy(table_hbm.at[idx_stage.at[g]],
                              rows_bufs[slot], sems[slot]).start()

    def wait_gather(slot):
        pltpu.make_async_copy(table_hbm.at[idx_stage.at[0]],
                              rows_bufs[slot], sems[slot]).wait()

    for s0 in range(NBUF):
        start_gather(s0, s0)

    three = jnp.bfloat16(3.0)
    lane = lax.iota(jnp.int32, 16)
    col_idx = [lane * 2 + (p % 2) + (p // 2) * 32 for p in range(2 * DP)]

    def compute_chunk(g, slot):
        rows = rows_bufs[slot]
        for n in range(C_NODES):
            acc = [jnp.zeros((16,), jnp.float32) for _ in range(2 * DP)]
            for e in range(DEG):
                base = n * DEG * ESZ + e * ESZ
                vsum = jnp.zeros((32,), jnp.bfloat16)
                msg = []
                for p in range(DP):
                    sl = pl.ds(p * 16, 16)
                    f0 = plsc.bitcast(rows[base + 0, sl], jnp.bfloat16)
                    f1 = plsc.bitcast(rows[base + 1, sl], jnp.bfloat16)
                    f2 = plsc.bitcast(rows[base + 2, sl], jnp.bfloat16)
                    s = f0 + f1 + f2
                    q = f0 * f0 + f1 * f1 + f2 * f2
                    vsum = vsum + (q * three - s * s)
                    msg.append(f0 * f1)
                va, vb = plsc.unpack(vsum, format=plsc.PackFormat.INTERLEAVED,
                                     preferred_element_type=jnp.float32)
                ev = jnp.sum(va + vb)
                evv = jnp.broadcast_to(ev, (16,))
                z = evv * wvec + bvec
                att = 1.0 / (1.0 + jnp.exp(-z))
                for p in range(DP):
                    ma, mb = plsc.unpack(msg[p],
                                         format=plsc.PackFormat.INTERLEAVED,
                                         preferred_element_type=jnp.float32)
                    acc[2 * p] = acc[2 * p] + att * ma
                    acc[2 * p + 1] = acc[2 * p + 1] + att * mb
            row = g * C_NODES + n
            rowv = jnp.broadcast_to(row, (16,)).astype(jnp.int32)
            for k in range(2 * DP):
                plsc.store_scatter(out_stage, [rowv, col_idx[k]], acc[k])

    def group(i, _):
        g0 = i * NBUF
        for slot in range(NBUF):
            g = g0 + slot
            wait_gather(slot)
            compute_chunk(g, slot)

            @pl.when(g + NBUF < CHUNKS_PER_W)
            def _():
                start_gather(g + NBUF, slot)
        return _

    lax.fori_loop(0, CHUNKS_PER_W // NBUF, group, None)
    pltpu.sync_copy(out_stage,
                    out_hbm.at[pl.ds(wid * NODES_PER_W, NODES_PER_W)])


@jax.jit
def _run(edge_idx, table_bf, w_vec, b_vec):
    mesh = plsc.VectorSubcoreMesh(core_axis_name="c", subcore_axis_name="s")
    f = pl.kernel(
        _body,
        out_type=jax.ShapeDtypeStruct((B_PAD, D), jnp.float32),
        mesh=mesh,
        compiler_params=pltpu.CompilerParams(needs_layout_passes=False,
                                             use_tc_tiling_on_sc=False),
        scratch_types=[
            pltpu.VMEM((CHUNKS_PER_W, ROWS_PER_CHUNK), jnp.int32),
            pltpu.VMEM((ROWS_PER_CHUNK, D // 2), jnp.int32),
            pltpu.VMEM((ROWS_PER_CHUNK, D // 2), jnp.int32),
            pltpu.VMEM((ROWS_PER_CHUNK, D // 2), jnp.int32),
            pltpu.VMEM((ROWS_PER_CHUNK, D // 2), jnp.int32),
            pltpu.VMEM((16,), jnp.float32),
            pltpu.VMEM((16,), jnp.float32),
            pltpu.VMEM((NODES_PER_W, D), jnp.float32),
            pltpu.SemaphoreType.DMA,
            pltpu.SemaphoreType.DMA,
            pltpu.SemaphoreType.DMA,
            pltpu.SemaphoreType.DMA,
        ],
    )
    return f(table_bf, edge_idx, w_vec, b_vec)


def kernel(nodes, edge_nodes, table, w_att_w, w_att_b):
    del nodes
    idx = edge_nodes.reshape(B, DEG * ESZ)
    idx = jnp.pad(idx, ((0, B_PAD - B), (0, 0)))
    idx = idx.reshape(B_PAD * DEG * ESZ // ROWS_PER_CHUNK, ROWS_PER_CHUNK)
    w_vec = jnp.full((16,), w_att_w[0, 0], jnp.float32)
    b_vec = jnp.full((16,), w_att_b[0], jnp.float32)
    table_i32 = lax.bitcast_convert_type(
        table.astype(jnp.bfloat16).reshape(N_NODES, D // 2, 2), jnp.int32)
    out = _run(idx, table_i32, w_vec, b_vec)
    return out[:B]
